# baseline (speedup 1.0000x reference)
import functools

import jax
import jax.numpy as jnp
from jax import lax
from jax.experimental import pallas as pl
from jax.experimental.pallas import tpu as pltpu
from jax.experimental.pallas import tpu_sc as plsc

_CH = 4
_NB = 12
_D = 6
_NW = 32


def _sc_body(seq, tail, kc, vc, ko, vo, ok, ov, *rest):
    shared = rest[0]
    tiles = rest[1:1 + _NB // 2]
    rsem = rest[1 + _NB // 2:1 + _NB // 2 + _NB]
    wsem = rest[1 + _NB // 2 + _NB:1 + _NB // 2 + 2 * _NB]
    c = lax.axis_index("c")
    s = lax.axis_index("s")
    bufs = [
        tiles[u // 2] if u % 2 == 0 else shared.at[s, u // 2]
        for u in range(_NB)
    ]
    w = s * 2 + c
    nq = _NW // 8
    head = w // nq
    q = lax.rem(w, nq)
    rows_w = seq // nq
    base = q * rows_w
    n = rows_w // _CH

    def stream(src, dst):
        def rd(i, u):
            return pltpu.make_async_copy(
                src.at[0, head, pl.ds(base + i * _CH, _CH)], bufs[u], rsem[u]
            )

        def wr(i, u):
            return pltpu.make_async_copy(
                bufs[u], dst.at[0, head, pl.ds(base + i * _CH, _CH)], wsem[u]
            )

        for i in range(_D):
            rd(i, i).start()
        for i in range(_NB):
            if i >= _D:
                wr(i - _D, i - _D).wait()
            if i + _D < n:
                rd(i + _D, (i + _D) % _NB).start()
            rd(i, i).wait()
            wr(i, i).start()

        n_steady = ((n - _NB - _D) // _NB) * _NB

        def step(j, carry):
            i0 = _NB + j * _NB
            for uu in range(_NB):
                i = i0 + uu
                wr(i - _D, (uu - _D) % _NB).wait()
                rd(i + _D, (uu + _D) % _NB).start()
                rd(i, uu).wait()
                wr(i, uu).start()
            return carry

        lax.fori_loop(0, n_steady // _NB, step, 0)

        for i in range(_NB + n_steady, n):
            wr(i - _D, (i - _D) % _NB).wait()
            if i + _D < n:
                rd(i + _D, (i + _D) % _NB).start()
            rd(i, i % _NB).wait()
            wr(i, i % _NB).start()
        for i in range(n - _D, n):
            wr(i, i % _NB).wait()

    stream(kc, ok)
    stream(vc, ov)

    @pl.when(q == 0)
    def _k_tail():
        for j in range(tail // _CH):
            pltpu.make_async_copy(
                ko.at[0, head, pl.ds(j * _CH, _CH)], bufs[j], rsem[j]
            ).start()
        for j in range(tail // _CH):
            pltpu.make_async_copy(
                ko.at[0, head, pl.ds(j * _CH, _CH)], bufs[j], rsem[j]
            ).wait()
            pltpu.make_async_copy(
                bufs[j], ok.at[0, head, pl.ds(seq + j * _CH, _CH)], wsem[j]
            ).start()
        for j in range(tail // _CH):
            pltpu.make_async_copy(
                bufs[j], ok.at[0, head, pl.ds(seq + j * _CH, _CH)], wsem[j]
            ).wait()

    @pl.when(q == 1)
    def _v_tail():
        for j in range(tail // _CH):
            pltpu.make_async_copy(
                vo.at[0, head, pl.ds(j * _CH, _CH)], bufs[j], rsem[j]
            ).start()
        for j in range(tail // _CH):
            pltpu.make_async_copy(
                vo.at[0, head, pl.ds(j * _CH, _CH)], bufs[j], rsem[j]
            ).wait()
            pltpu.make_async_copy(
                bufs[j], ov.at[0, head, pl.ds(seq + j * _CH, _CH)], wsem[j]
            ).start()
        for j in range(tail // _CH):
            pltpu.make_async_copy(
                bufs[j], ov.at[0, head, pl.ds(seq + j * _CH, _CH)], wsem[j]
            ).wait()


def kernel(k_cache, v_cache, k_out, v_out):
    b, h, seq, n, d = k_cache.shape
    tail = k_out.shape[2]
    nq = _NW // 8
    rows_w = seq // nq
    assert b * h == 8 and seq % nq == 0 and rows_w % _CH == 0
    assert rows_w // _CH >= _NB + 2 * _D
    assert tail % _CH == 0 and tail // _CH <= _NB
    out_sds = jax.ShapeDtypeStruct((b, h, seq + tail, n, d), k_cache.dtype)
    mesh = plsc.VectorSubcoreMesh(core_axis_name="c", subcore_axis_name="s")
    fn = pl.kernel(
        functools.partial(_sc_body, seq, tail),
        mesh=mesh,
        out_type=[out_sds, out_sds],
        scratch_types=(
            [pltpu.VMEM_SHARED((16, _NB // 2, _CH, n, d), k_cache.dtype)]
            + [pltpu.VMEM((_CH, n, d), k_cache.dtype) for _ in range(_NB // 2)]
            + [pltpu.SemaphoreType.DMA] * (2 * _NB)
        ),
    )
    k_new, v_new = fn(k_cache, v_cache, k_out, v_out)
    return (k_new, v_new)


# speedup vs baseline: 1.0193x; 1.0193x over previous
import functools

import jax
import jax.numpy as jnp
from jax import lax
from jax.experimental import pallas as pl
from jax.experimental.pallas import tpu as pltpu
from jax.experimental.pallas import tpu_sc as plsc

_CH = 8
_NB = 6
_D = 3
_NW = 32


def _sc_body(seq, tail, kc, vc, ko, vo, ok, ov, *rest):
    shared = rest[0]
    rsem = rest[1:1 + _NB]
    wsem = rest[1 + _NB:1 + 2 * _NB]
    c = lax.axis_index("c")
    s = lax.axis_index("s")
    bufs = [shared.at[s, u] for u in range(_NB)]
    w = s * 2 + c
    nq = _NW // 8
    head = w // nq
    q = lax.rem(w, nq)
    rows_w = seq // nq
    base = q * rows_w
    n = rows_w // _CH

    def stream(src, dst):
        def rd(i, u):
            return pltpu.make_async_copy(
                src.at[0, head, pl.ds(base + i * _CH, _CH)], bufs[u], rsem[u]
            )

        def wr(i, u):
            return pltpu.make_async_copy(
                bufs[u], dst.at[0, head, pl.ds(base + i * _CH, _CH)], wsem[u]
            )

        for i in range(_D):
            rd(i, i).start()
        for i in range(_NB):
            if i >= _D:
                wr(i - _D, i - _D).wait()
            if i + _D < n:
                rd(i + _D, (i + _D) % _NB).start()
            rd(i, i).wait()
            wr(i, i).start()

        n_steady = ((n - _NB - _D) // _NB) * _NB

        def step(j, carry):
            i0 = _NB + j * _NB
            for uu in range(_NB):
                i = i0 + uu
                wr(i - _D, (uu - _D) % _NB).wait()
                rd(i + _D, (uu + _D) % _NB).start()
                rd(i, uu).wait()
                wr(i, uu).start()
            return carry

        lax.fori_loop(0, n_steady // _NB, step, 0)

        for i in range(_NB + n_steady, n):
            wr(i - _D, (i - _D) % _NB).wait()
            if i + _D < n:
                rd(i + _D, (i + _D) % _NB).start()
            rd(i, i % _NB).wait()
            wr(i, i % _NB).start()
        for i in range(n - _D, n):
            wr(i, i % _NB).wait()

    stream(kc, ok)
    stream(vc, ov)

    @pl.when(q == 0)
    def _k_tail():
        for j in range(tail // _CH):
            pltpu.make_async_copy(
                ko.at[0, head, pl.ds(j * _CH, _CH)], bufs[j], rsem[j]
            ).start()
        for j in range(tail // _CH):
            pltpu.make_async_copy(
                ko.at[0, head, pl.ds(j * _CH, _CH)], bufs[j], rsem[j]
            ).wait()
            pltpu.make_async_copy(
                bufs[j], ok.at[0, head, pl.ds(seq + j * _CH, _CH)], wsem[j]
            ).start()
        for j in range(tail // _CH):
            pltpu.make_async_copy(
                bufs[j], ok.at[0, head, pl.ds(seq + j * _CH, _CH)], wsem[j]
            ).wait()

    @pl.when(q == 1)
    def _v_tail():
        for j in range(tail // _CH):
            pltpu.make_async_copy(
                vo.at[0, head, pl.ds(j * _CH, _CH)], bufs[j], rsem[j]
            ).start()
        for j in range(tail // _CH):
            pltpu.make_async_copy(
                vo.at[0, head, pl.ds(j * _CH, _CH)], bufs[j], rsem[j]
            ).wait()
            pltpu.make_async_copy(
                bufs[j], ov.at[0, head, pl.ds(seq + j * _CH, _CH)], wsem[j]
            ).start()
        for j in range(tail // _CH):
            pltpu.make_async_copy(
                bufs[j], ov.at[0, head, pl.ds(seq + j * _CH, _CH)], wsem[j]
            ).wait()


def kernel(k_cache, v_cache, k_out, v_out):
    b, h, seq, n, d = k_cache.shape
    tail = k_out.shape[2]
    nq = _NW // 8
    rows_w = seq // nq
    assert b * h == 8 and seq % nq == 0 and rows_w % _CH == 0
    assert rows_w // _CH >= _NB + 2 * _D
    assert tail % _CH == 0 and tail // _CH <= _NB
    out_sds = jax.ShapeDtypeStruct((b, h, seq + tail, n, d), k_cache.dtype)
    mesh = plsc.VectorSubcoreMesh(core_axis_name="c", subcore_axis_name="s")
    fn = pl.kernel(
        functools.partial(_sc_body, seq, tail),
        mesh=mesh,
        out_type=[out_sds, out_sds],
        scratch_types=(
            [pltpu.VMEM_SHARED((16, _NB, _CH, n, d), k_cache.dtype)]
            + [pltpu.SemaphoreType.DMA] * (2 * _NB)
        ),
    )
    k_new, v_new = fn(k_cache, v_cache, k_out, v_out)
    return (k_new, v_new)


# speedup vs baseline: 1.0362x; 1.0166x over previous
import functools

import jax
import jax.numpy as jnp
from jax import lax
from jax.experimental import pallas as pl
from jax.experimental.pallas import tpu as pltpu
from jax.experimental.pallas import tpu_sc as plsc

_CH = 4
_NB = 8
_D = 4
_NW = 32


def _sc_body(seq, tail, kc, vc, ko, vo, ok, ov, *rest):
    shared = rest[0]
    rsem = rest[1:1 + _NB]
    wsem = rest[1 + _NB:1 + 2 * _NB]
    c = lax.axis_index("c")
    s = lax.axis_index("s")
    bufs = [shared.at[s, u] for u in range(_NB)]
    w = s * 2 + c
    nq = _NW // 8
    head = w // nq
    q = lax.rem(w, nq)
    rows_w = seq // nq
    base = q * rows_w
    n = rows_w // _CH

    ng = 2 * n

    def rd(g, u):
        src = kc if u % 2 == 0 else vc
        m = (g - (u % 2)) // 2
        return pltpu.make_async_copy(
            src.at[0, head, pl.ds(base + m * _CH, _CH)], bufs[u], rsem[u]
        )

    def wr(g, u):
        dst = ok if u % 2 == 0 else ov
        m = (g - (u % 2)) // 2
        return pltpu.make_async_copy(
            bufs[u], dst.at[0, head, pl.ds(base + m * _CH, _CH)], wsem[u]
        )

    for g in range(_D):
        rd(g, g).start()
    for g in range(_NB):
        if g >= _D:
            wr(g - _D, g - _D).wait()
        if g + _D < ng:
            rd(g + _D, (g + _D) % _NB).start()
        rd(g, g).wait()
        wr(g, g).start()

    n_steady = ((ng - _NB - _D) // _NB) * _NB

    def step(j, carry):
        g0 = _NB + j * _NB
        for uu in range(_NB):
            g = g0 + uu
            wr(g - _D, (uu - _D) % _NB).wait()
            rd(g + _D, (uu + _D) % _NB).start()
            rd(g, uu).wait()
            wr(g, uu).start()
        return carry

    lax.fori_loop(0, n_steady // _NB, step, 0)

    for g in range(_NB + n_steady, ng):
        wr(g - _D, (g - _D) % _NB).wait()
        if g + _D < ng:
            rd(g + _D, (g + _D) % _NB).start()
        rd(g, g % _NB).wait()
        wr(g, g % _NB).start()
    for g in range(ng - _D, ng):
        wr(g, g % _NB).wait()

    @pl.when(q == 0)
    def _k_tail():
        for j in range(tail // _CH):
            pltpu.make_async_copy(
                ko.at[0, head, pl.ds(j * _CH, _CH)], bufs[j], rsem[j]
            ).start()
        for j in range(tail // _CH):
            pltpu.make_async_copy(
                ko.at[0, head, pl.ds(j * _CH, _CH)], bufs[j], rsem[j]
            ).wait()
            pltpu.make_async_copy(
                bufs[j], ok.at[0, head, pl.ds(seq + j * _CH, _CH)], wsem[j]
            ).start()
        for j in range(tail // _CH):
            pltpu.make_async_copy(
                bufs[j], ok.at[0, head, pl.ds(seq + j * _CH, _CH)], wsem[j]
            ).wait()

    @pl.when(q == 1)
    def _v_tail():
        for j in range(tail // _CH):
            pltpu.make_async_copy(
                vo.at[0, head, pl.ds(j * _CH, _CH)], bufs[j], rsem[j]
            ).start()
        for j in range(tail // _CH):
            pltpu.make_async_copy(
                vo.at[0, head, pl.ds(j * _CH, _CH)], bufs[j], rsem[j]
            ).wait()
            pltpu.make_async_copy(
                bufs[j], ov.at[0, head, pl.ds(seq + j * _CH, _CH)], wsem[j]
            ).start()
        for j in range(tail // _CH):
            pltpu.make_async_copy(
                bufs[j], ov.at[0, head, pl.ds(seq + j * _CH, _CH)], wsem[j]
            ).wait()


def kernel(k_cache, v_cache, k_out, v_out):
    b, h, seq, n, d = k_cache.shape
    tail = k_out.shape[2]
    nq = _NW // 8
    rows_w = seq // nq
    assert b * h == 8 and seq % nq == 0 and rows_w % _CH == 0
    assert 2 * (rows_w // _CH) >= _NB + 2 * _D
    assert tail % _CH == 0 and tail // _CH <= _NB
    out_sds = jax.ShapeDtypeStruct((b, h, seq + tail, n, d), k_cache.dtype)
    mesh = plsc.VectorSubcoreMesh(core_axis_name="c", subcore_axis_name="s")
    fn = pl.kernel(
        functools.partial(_sc_body, seq, tail),
        mesh=mesh,
        out_type=[out_sds, out_sds],
        scratch_types=(
            [pltpu.VMEM_SHARED((16, _NB, _CH, n, d), k_cache.dtype)]
            + [pltpu.SemaphoreType.DMA] * (2 * _NB)
        ),
    )
    k_new, v_new = fn(k_cache, v_cache, k_out, v_out)
    return (k_new, v_new)
